# Initial kernel scaffold; baseline (speedup 1.0000x reference)
import jax
import jax.numpy as jnp
from jax.experimental import pallas as pl


def kernel(det_boxes, det_scores, det_labels, boxes, labels):
    raise NotImplementedError("write your pallas kernel here")


# speedup vs baseline: 5.9758x; 5.9758x over previous
import functools

import jax
import jax.numpy as jnp
from jax import lax
from jax.experimental import pallas as pl

N_DET = 5000
N_PAD = 5120
N_OBJ = 64
THRESHOLD = 0.5
NEG_POS_RATIO = 3
ALPHA = 1.0


def _loss_kernel(dbt_ref, ds_ref, b_ref, lab_ref, out_ref):
    dx1 = dbt_ref[0:1, :]
    dy1 = dbt_ref[1:2, :]
    dx2 = dbt_ref[2:3, :]
    dy2 = dbt_ref[3:4, :]
    bx1 = b_ref[:, 0:1]
    by1 = b_ref[:, 1:2]
    bx2 = b_ref[:, 2:3]
    by2 = b_ref[:, 3:4]

    lx = jnp.maximum(bx1, dx1)
    ly = jnp.maximum(by1, dy1)
    ux = jnp.minimum(bx2, dx2)
    uy = jnp.minimum(by2, dy2)
    iw = jnp.maximum(ux - lx, 0.0)
    ih = jnp.maximum(uy - ly, 0.0)
    inter = iw * ih
    a1 = (bx2 - bx1) * (by2 - by1)
    a2 = (dx2 - dx1) * (dy2 - dy1)
    ov = inter / (a1 + a2 - inter)

    colid = lax.broadcasted_iota(jnp.int32, (N_OBJ, N_PAD), 1)
    rowid = lax.broadcasted_iota(jnp.int32, (N_OBJ, N_PAD), 0)
    valid = colid < N_DET
    ov = jnp.where(valid, ov, -1.0)

    ov_det = jnp.max(ov, axis=0, keepdims=True)
    obj_det = jnp.min(jnp.where(ov == ov_det, rowid, N_OBJ),
                      axis=0, keepdims=True)

    row_max = jnp.max(ov, axis=1, keepdims=True)
    det_obj = jnp.min(jnp.where(ov == row_max, colid, N_PAD),
                      axis=1, keepdims=True)

    eq_scat = det_obj == colid
    scat = jnp.max(jnp.where(eq_scat, rowid, -1), axis=0, keepdims=True)
    hit = scat >= 0
    obj_det = jnp.where(hit, scat, obj_det)
    ov_det = jnp.where(hit, 1.0, ov_det)

    eq_obj = rowid == obj_det
    lab_det = jnp.max(jnp.where(eq_obj, lab_ref[:, 0:1], 0),
                      axis=0, keepdims=True)
    lab_det = jnp.where(ov_det < THRESHOLD, 0, lab_det)

    pos = lab_det != 0
    posf = pos.astype(jnp.float32)
    n_pos = jnp.sum(posf)

    loc = jnp.abs(dx1 - jnp.sum(jnp.where(eq_obj, bx1, 0.0), axis=0,
                                keepdims=True))
    loc = loc + jnp.abs(dy1 - jnp.sum(jnp.where(eq_obj, by1, 0.0), axis=0,
                                      keepdims=True))
    loc = loc + jnp.abs(dx2 - jnp.sum(jnp.where(eq_obj, bx2, 0.0), axis=0,
                                      keepdims=True))
    loc = loc + jnp.abs(dy2 - jnp.sum(jnp.where(eq_obj, by2, 0.0), axis=0,
                                      keepdims=True))
    loc_loss = jnp.sum(loc * posf) / (n_pos * 4.0)

    conf = 1.0 - ds_ref[0:1, :]
    conf_pos_sum = jnp.sum(conf * posf)
    conf_neg = jnp.where(pos, 0.0, conf)

    kn = NEG_POS_RATIO * jnp.sum(pos.astype(jnp.int32))
    bits = lax.bitcast_convert_type(conf_neg, jnp.int32)

    def body(_, carry):
        lo, hi = carry
        mid = (lo + hi) // 2
        cnt = jnp.sum((bits >= mid).astype(jnp.int32))
        ok = cnt >= kn
        return jnp.where(ok, mid, lo), jnp.where(ok, hi, mid)

    lo, _ = lax.fori_loop(0, 31, body, (jnp.int32(0), jnp.int32(0x3F800001)))
    gt = bits > lo
    cnt_gt = jnp.sum(gt.astype(jnp.int32))
    sum_gt = jnp.sum(jnp.where(gt, conf_neg, 0.0))
    tval = lax.bitcast_convert_type(lo, jnp.float32)
    conf_hard_sum = sum_gt + (kn - cnt_gt).astype(jnp.float32) * tval

    conf_loss = (conf_hard_sum + conf_pos_sum) / n_pos
    out_ref[...] = jnp.reshape(conf_loss + ALPHA * loc_loss, (1, 1))


@functools.partial(jax.jit, static_argnames=("interpret",))
def kernel(det_boxes, det_scores, det_labels, boxes, labels, interpret=False):
    del det_labels
    db = det_boxes[0].astype(jnp.float32)
    dbt = jnp.pad(db, ((0, N_PAD - N_DET), (0, 0))).T
    ds = jnp.pad(det_scores[0].astype(jnp.float32),
                 (0, N_PAD - N_DET), constant_values=1.0)[None, :]
    b = boxes[0].astype(jnp.float32)
    lab = labels[0].astype(jnp.int32)[:, None]

    out = pl.pallas_call(
        _loss_kernel,
        out_shape=jax.ShapeDtypeStruct((1, 1), jnp.float32),
        interpret=interpret,
    )(dbt, ds, b, lab)
    return out[0, 0]
